# Initial kernel scaffold; baseline (speedup 1.0000x reference)
import jax
import jax.numpy as jnp
from jax.experimental import pallas as pl


def kernel(table, params, item_ids, seq1_src, seq1_dst, seq2_src, seq2_dst, seq3_src, seq3_dst, up1_src, up1_dst, down1_src, down1_dst, up2_src, up2_dst, down2_src, down2_dst):
    raise NotImplementedError("write your pallas kernel here")


# speedup vs baseline: 28.2864x; 28.2864x over previous
import functools

import jax
import jax.numpy as jnp
from jax import lax
from jax.experimental import pallas as pl
from jax.experimental.pallas import tpu as pltpu

EMB = 128
HID = 128
H = 8
D = HID // H
N1, N2, N3 = 50000, 25000, 12500


def _readout_sum_kernel(x_ref, w1_ref, b1_ref, acc_ref, *, nblocks):
    i = pl.program_id(0)

    @pl.when(i == 0)
    def _():
        acc_ref[...] = jnp.zeros_like(acc_ref)

    t = jnp.maximum(
        jnp.dot(x_ref[...], w1_ref[...], preferred_element_type=jnp.float32)
        + b1_ref[...],
        0.0,
    )
    acc_ref[...] += jnp.sum(t, axis=0, keepdims=True)


def _readout_sum(x, w1, b1, block=1000):
    n = x.shape[0]
    nblocks = n // block
    assert nblocks * block == n
    return pl.pallas_call(
        functools.partial(_readout_sum_kernel, nblocks=nblocks),
        grid=(nblocks,),
        in_specs=[
            pl.BlockSpec((block, HID), lambda i: (i, 0)),
            pl.BlockSpec((HID, HID), lambda i: (0, 0)),
            pl.BlockSpec((1, HID), lambda i: (0, 0)),
        ],
        out_specs=pl.BlockSpec((1, HID), lambda i: (0, 0)),
        out_shape=jax.ShapeDtypeStruct((1, HID), jnp.float32),
    )(x, w1, b1.reshape(1, HID))


def _gat_edges(f, el, er, src, dst, n_dst):
    e = el[src] + (er[dst] if er is not None else 0.0)
    w = jnp.exp(jnp.where(e >= 0, e, 0.2 * e))
    den = jax.ops.segment_sum(w, dst, num_segments=n_dst)
    msg = f[src].reshape(-1, H, D) * w[..., None]
    num = jax.ops.segment_sum(msg.reshape(-1, HID), dst, num_segments=n_dst)
    return num, den


def _finish(num, den, extra, p_ro, n):
    rep = jnp.repeat(den, D, axis=1)
    o = jnp.where(rep > 0, num / jnp.where(rep > 0, rep, 1.0), 0.0) + extra
    s = _readout_sum(o, p_ro['W1'], p_ro['b1'])
    return (s / n) @ p_ro['W2'] + p_ro['b2']


def kernel(table, params, item_ids, seq1_src, seq1_dst, seq2_src, seq2_dst,
           seq3_src, seq3_dst, up1_src, up1_dst, down1_src, down1_dst,
           up2_src, up2_dst, down2_src, down2_dst):
    p1, pu = params['seq1'], params['up1']
    h1 = jnp.take(table, item_ids, axis=0)

    f1 = h1 @ p1['W']
    el1 = (f1.reshape(-1, H, D) * p1['al'][None]).sum(-1)
    er1 = (f1.reshape(-1, H, D) * p1['ar'][None]).sum(-1)
    num1, den1 = _gat_edges(f1, el1, er1, seq1_src, seq1_dst, N1)
    extra1 = h1 + (p1['b'] + params['down1']['b']).reshape(1, HID)
    r1 = _finish(num1, den1, extra1, params['ro1'], N1)

    fu = h1 @ pu['W']
    elu = (fu.reshape(-1, H, D) * pu['al'][None]).sum(-1)
    numu, denu = _gat_edges(fu, elu, None, up1_src, up1_dst, N2)
    extra2 = (params['seq2']['b'] + pu['b'] + params['down2']['b']).reshape(1, HID)
    r2 = _finish(numu, denu, extra2, params['ro2'], N2)

    c3 = (params['seq3']['b'] + params['up2']['b']).reshape(1, HID)
    p3 = params['ro3']
    r3 = jnp.maximum(c3 @ p3['W1'] + p3['b1'], 0.0) @ p3['W2'] + p3['b2']

    stack = jnp.concatenate([r1, r2, r3], axis=0)
    w = jax.nn.softmax(params['gw'])
    fused = (stack * w[:, None]).sum(axis=0)
    return fused, stack, w


# speedup vs baseline: 31.5677x; 1.1160x over previous
import functools

import jax
import jax.numpy as jnp
from jax import lax
from jax.experimental import pallas as pl
from jax.experimental.pallas import tpu as pltpu
from jax.experimental.pallas import tpu_sc as plsc

EMB = 128
HID = 128
H = 8
D = HID // H
N1, N2, N3 = 50000, 25000, 12500

NC, NS, LANES = 2, 16, 16
NW = NC * NS
FW = 144
NSEG = FW // LANES
BAT = 128
N1P = 50176
N2P = 25088
E1P = 200704
E2P = 102400

_SC_PARAMS = pltpu.CompilerParams(use_tc_tiling_on_sc=False)


def _emb_gather(table, idx_pad):
    per_w = N1P // NW
    sub = 224
    mesh = plsc.VectorSubcoreMesh(core_axis_name="c", subcore_axis_name="s")

    @functools.partial(
        pl.kernel, mesh=mesh,
        out_type=jax.ShapeDtypeStruct((N1P, EMB), jnp.float32),
        scratch_types=[
            pltpu.VMEM((per_w,), jnp.int32),
            pltpu.VMEM((sub, EMB), jnp.float32),
            pltpu.SemaphoreType.DMA,
        ],
        compiler_params=_SC_PARAMS,
    )
    def k(idx_hbm, table_hbm, out_hbm, idx_v, rows_v, sem):
        wid = lax.axis_index("s") * NC + lax.axis_index("c")
        base = wid * per_w
        pltpu.sync_copy(idx_hbm.at[pl.ds(base, per_w)], idx_v)
        for j in range(per_w // sub):
            pltpu.async_copy(
                table_hbm.at[idx_v.at[pl.ds(j * sub, sub)]], rows_v, sem
            ).wait()
            pltpu.sync_copy(rows_v, out_hbm.at[pl.ds(base + j * sub, sub)])

    return k(idx_pad, table)


def _prep_kernel(h_ref, w1_ref, a1_ref, wu_ref, au_ref, s_ref,
                 f1_ref, er_ref, fu_ref):
    h = h_ref[...]
    z8 = jnp.zeros((h.shape[0], 8), jnp.float32)
    f1 = jnp.dot(h, w1_ref[...], preferred_element_type=jnp.float32)
    el = jnp.dot(f1 * a1_ref[0:1, :], s_ref[...],
                 preferred_element_type=jnp.float32)
    er = jnp.dot(f1 * a1_ref[1:2, :], s_ref[...],
                 preferred_element_type=jnp.float32)
    f1_ref[...] = jnp.concatenate([f1, el, z8], axis=1)
    er_ref[...] = jnp.concatenate([er, z8], axis=1)
    fu = jnp.dot(h, wu_ref[...], preferred_element_type=jnp.float32)
    elu = jnp.dot(fu * au_ref[0:1, :], s_ref[...],
                  preferred_element_type=jnp.float32)
    fu_ref[...] = jnp.concatenate([fu, elu, z8], axis=1)


def _prep(h1p, w1, a1, wu, au, smat, block=1024):
    nblk = N1P // block
    return pl.pallas_call(
        _prep_kernel,
        grid=(nblk,),
        in_specs=[
            pl.BlockSpec((block, EMB), lambda i: (i, 0)),
            pl.BlockSpec((EMB, HID), lambda i: (0, 0)),
            pl.BlockSpec((2, HID), lambda i: (0, 0)),
            pl.BlockSpec((EMB, HID), lambda i: (0, 0)),
            pl.BlockSpec((2, HID), lambda i: (0, 0)),
            pl.BlockSpec((HID, 8), lambda i: (0, 0)),
        ],
        out_specs=[
            pl.BlockSpec((block, FW), lambda i: (i, 0)),
            pl.BlockSpec((block, 16), lambda i: (i, 0)),
            pl.BlockSpec((block, FW), lambda i: (i, 0)),
        ],
        out_shape=[
            jax.ShapeDtypeStruct((N1P, FW), jnp.float32),
            jax.ShapeDtypeStruct((N1P, 16), jnp.float32),
            jax.ShapeDtypeStruct((N1P, FW), jnp.float32),
        ],
    )(h1p, w1, a1, wu, au, smat)


def _sc_gat(fflat, ertab, src_pad, dst_pad, zstripe, np_dst, has_er):
    ep = src_pad.shape[0]
    epb = ep // NW
    nbat = epb // BAT
    stripe = np_dst // NS
    mesh = plsc.VectorSubcoreMesh(core_axis_name="c", subcore_axis_name="s")

    scratch = [
        pltpu.VMEM((epb,), jnp.int32),
        pltpu.VMEM((epb,), jnp.int32),
        pltpu.VMEM((BAT, LANES), jnp.float32),
        pltpu.VMEM((BAT, LANES), jnp.float32),
        pltpu.VMEM((BAT, LANES), jnp.float32),
        pltpu.VMEM((BAT,), jnp.int32),
        pltpu.VMEM((BAT,), jnp.int32),
        pltpu.VMEM((BAT,), jnp.int32),
        pltpu.VMEM_SHARED((np_dst, LANES), jnp.float32),
        pltpu.SemaphoreType.DMA,
        pltpu.SemaphoreType.DMA,
    ]

    @functools.partial(
        pl.kernel, mesh=mesh,
        out_type=[
            jax.ShapeDtypeStruct((NC, np_dst, NSEG, LANES), jnp.float32),
            jax.ShapeDtypeStruct((ep, LANES), jnp.float32),
        ],
        scratch_types=scratch,
        compiler_params=_SC_PARAMS,
    )
    def k(f_hbm, er_hbm, src_hbm, dst_hbm, z_hbm, out_hbm, w_hbm,
          srcv, dstv, arows, brows, wrows, gI, gI2, sI, acc, semA, semB):
        cid = lax.axis_index("c")
        sid = lax.axis_index("s")
        wid = sid * NC + cid
        ebase = wid * epb
        lane = lax.iota(jnp.int32, LANES)

        pltpu.sync_copy(src_hbm.at[pl.ds(ebase, epb)], srcv)
        pltpu.sync_copy(dst_hbm.at[pl.ds(ebase, epb)], dstv)

        def pass_body(p, _):
            pltpu.sync_copy(z_hbm, acc.at[pl.ds(sid * stripe, stripe)])
            plsc.subcore_barrier()

            def bbody(j, _):
                jb = j * BAT

                def gidx(g, _):
                    off = jb + g * LANES
                    sv = srcv[pl.ds(off, LANES)]
                    dv = dstv[pl.ds(off, LANES)]
                    gcol = jnp.where(p == 0, 8, p - 1)
                    gI[pl.ds(g * LANES, LANES)] = sv * NSEG + gcol
                    sI[pl.ds(g * LANES, LANES)] = jnp.where(
                        dv >= 0, dv, np_dst - 1)
                    return 0
                lax.fori_loop(0, BAT // LANES, gidx, 0)

                cpA = pltpu.async_copy(f_hbm.at[gI], arows, semA)
                if has_er:
                    @pl.when(p == 0)
                    def _():
                        def eidx(g, _):
                            off = jb + g * LANES
                            dv = dstv[pl.ds(off, LANES)]
                            gI2[pl.ds(g * LANES, LANES)] = jnp.where(
                                dv >= 0, dv, 0)
                            return 0
                        lax.fori_loop(0, BAT // LANES, eidx, 0)
                        pltpu.async_copy(er_hbm.at[gI2], brows, semB).wait()

                @pl.when(p > 0)
                def _():
                    pltpu.async_copy(w_hbm.at[pl.ds(ebase + jb, BAT)],
                                     wrows, semB).wait()
                cpA.wait()

                @pl.when(p == 0)
                def _():
                    def ebody(e, _):
                        elv = arows[e, pl.ds(0, LANES)]
                        sv = elv + brows[e, pl.ds(0, LANES)] if has_er else elv
                        sv = jnp.where(sv >= 0.0, sv, 0.2 * sv)
                        wrows[e, pl.ds(0, LANES)] = jnp.exp(sv)
                        return 0
                    lax.fori_loop(0, BAT, ebody, 0)
                    pltpu.sync_copy(wrows, acc.at[sI], add=True)
                    pltpu.sync_copy(wrows,
                                    w_hbm.at[pl.ds(ebase + jb, BAT)])

                for hh in range(1, NSEG):
                    @pl.when(p == hh)
                    def _(hh=hh):
                        def ebody(e, _):
                            wrow = wrows[e, pl.ds(0, LANES)]
                            frow = arows[e, pl.ds(0, LANES)]
                            arows[e, pl.ds(0, LANES)] = frow * wrow[hh - 1]
                            return 0
                        lax.fori_loop(0, BAT, ebody, 0)
                        pltpu.sync_copy(arows, acc.at[sI], add=True)
                return 0
            lax.fori_loop(0, nbat, bbody, 0)
            plsc.subcore_barrier()

            seg = jnp.where(p == 0, NSEG - 1, p - 1)
            pltpu.sync_copy(acc.at[pl.ds(sid * stripe, stripe)],
                            out_hbm.at[cid, pl.ds(sid * stripe, stripe), seg])
            plsc.subcore_barrier()
            return 0
        lax.fori_loop(0, NSEG, pass_body, 0)

    return k(fflat, ertab, src_pad, dst_pad, zstripe)[0]


def _readout_kernel(*refs, nvalid, nblk, blk, with_res):
    if with_res:
        (o0_ref, o1_ref, hres_ref, rep_ref, w1_ref, b1_ref, w2_ref, b2_ref,
         bias_ref, out_ref) = refs
    else:
        (o0_ref, o1_ref, rep_ref, w1_ref, b1_ref, w2_ref, b2_ref,
         bias_ref, out_ref) = refs
        hres_ref = None
    b = pl.program_id(0)

    @pl.when(b == 0)
    def _():
        out_ref[...] = jnp.zeros_like(out_ref)

    num = o0_ref[:, :HID] + o1_ref[:, :HID]
    den8 = o0_ref[:, HID:HID + 8] + o1_ref[:, HID:HID + 8]
    den = jnp.dot(den8, rep_ref[...], preferred_element_type=jnp.float32)
    o = jnp.where(den > 0, num / jnp.where(den > 0, den, 1.0), 0.0)
    if with_res:
        o = o + hres_ref[...]
    o = o + bias_ref[...]
    t = jnp.maximum(
        jnp.dot(o, w1_ref[...], preferred_element_type=jnp.float32)
        + b1_ref[...], 0.0)
    rid = b * blk + lax.broadcasted_iota(jnp.int32, t.shape, 0)
    t = jnp.where(rid < nvalid, t, 0.0)
    out_ref[...] += jnp.sum(t, axis=0, keepdims=True)

    @pl.when(b == nblk - 1)
    def _():
        out_ref[...] = (
            jnp.dot(out_ref[...] * (1.0 / nvalid), w2_ref[...],
                    preferred_element_type=jnp.float32) + b2_ref[...])


def _readout(parts, hres, rep, p_ro, bias_row, np_dst, nvalid, blk):
    o2 = parts.reshape(NC, np_dst, FW)
    nblk = np_dst // blk
    in_specs = [
        pl.BlockSpec((blk, FW), lambda b: (b, 0)),
        pl.BlockSpec((blk, FW), lambda b: (b, 0)),
        pl.BlockSpec((blk, HID), lambda b: (b, 0)),
        pl.BlockSpec((8, HID), lambda b: (0, 0)),
        pl.BlockSpec((HID, HID), lambda b: (0, 0)),
        pl.BlockSpec((1, HID), lambda b: (0, 0)),
        pl.BlockSpec((HID, EMB), lambda b: (0, 0)),
        pl.BlockSpec((1, EMB), lambda b: (0, 0)),
        pl.BlockSpec((1, HID), lambda b: (0, 0)),
    ]
    args = [o2[0], o2[1], hres, rep, p_ro['W1'], p_ro['b1'].reshape(1, HID),
            p_ro['W2'], p_ro['b2'].reshape(1, EMB), bias_row]
    with_res = hres is not None
    if not with_res:
        in_specs.pop(2)
        args.pop(2)
    return pl.pallas_call(
        functools.partial(_readout_kernel, nvalid=nvalid, nblk=nblk, blk=blk,
                          with_res=with_res),
        grid=(nblk,),
        in_specs=in_specs,
        out_specs=pl.BlockSpec((1, EMB), lambda b: (0, 0)),
        out_shape=jax.ShapeDtypeStruct((1, EMB), jnp.float32),
    )(*args)


def _ro_row_kernel(x_ref, w1_ref, b1_ref, w2_ref, b2_ref, out_ref):
    t = jnp.maximum(
        jnp.dot(x_ref[...], w1_ref[...], preferred_element_type=jnp.float32)
        + b1_ref[...], 0.0)
    out_ref[...] = (jnp.dot(t, w2_ref[...], preferred_element_type=jnp.float32)
                    + b2_ref[...])


def _ro_row(x_row, p_ro):
    x8 = jnp.broadcast_to(x_row, (8, HID))
    out = pl.pallas_call(
        _ro_row_kernel,
        out_shape=jax.ShapeDtypeStruct((8, EMB), jnp.float32),
    )(x8, p_ro['W1'], p_ro['b1'].reshape(1, HID), p_ro['W2'],
      p_ro['b2'].reshape(1, EMB))
    return out[0:1]


def kernel(table, params, item_ids, seq1_src, seq1_dst, seq2_src, seq2_dst,
           seq3_src, seq3_dst, up1_src, up1_dst, down1_src, down1_dst,
           up2_src, up2_dst, down2_src, down2_dst):
    p1, pu = params['seq1'], params['up1']

    idx_pad = jnp.pad(item_ids.astype(jnp.int32), (0, N1P - N1))
    h1p = _emb_gather(table, idx_pad)

    a1 = jnp.stack([p1['al'].reshape(HID), p1['ar'].reshape(HID)])
    au = jnp.stack([pu['al'].reshape(HID), pu['ar'].reshape(HID)])
    smat = (jnp.arange(HID)[:, None] // D ==
            jnp.arange(8)[None, :]).astype(jnp.float32)
    f1mat, ermat, fumat = _prep(h1p, p1['W'], a1, pu['W'], au, smat)
    f1flat = f1mat.reshape(N1P * NSEG, LANES)
    fuflat = fumat.reshape(N1P * NSEG, LANES)

    def pad_edges(s, d, ep):
        e = s.shape[0]
        s = jnp.pad(s.astype(jnp.int32), (0, ep - e))
        d = jnp.pad(d.astype(jnp.int32), (0, ep - e), constant_values=-1)
        return s, d

    s1, d1 = pad_edges(seq1_src, seq1_dst, E1P)
    su, du = pad_edges(up1_src, up1_dst, E2P)

    z1 = jnp.zeros((N1P // NS, LANES), jnp.float32)
    z2 = jnp.zeros((N2P // NS, LANES), jnp.float32)
    parts1 = _sc_gat(f1flat, ermat, s1, d1, z1, N1P, has_er=True)
    partsu = _sc_gat(fuflat, ermat, su, du, z2, N2P, has_er=False)

    rep = (jnp.arange(8)[:, None] ==
           jnp.arange(HID)[None, :] // D).astype(jnp.float32)
    bias1 = (p1['b'] + params['down1']['b']).reshape(1, HID)
    bias2 = (params['seq2']['b'] + pu['b'] + params['down2']['b']).reshape(1, HID)

    r1 = _readout(parts1, h1p, rep, params['ro1'], bias1, N1P, N1, blk=1024)
    r2 = _readout(partsu, None, rep, params['ro2'], bias2, N2P, N2, blk=512)

    c3 = (params['seq3']['b'] + params['up2']['b']).reshape(1, HID)
    r3 = _ro_row(c3, params['ro3'])

    stack = jnp.concatenate([r1, r2, r3], axis=0)
    w = jax.nn.softmax(params['gw'])
    fused = (stack * w[:, None]).sum(axis=0)
    return fused, stack, w


# speedup vs baseline: 44.2498x; 1.4017x over previous
import functools

import jax
import jax.numpy as jnp
from jax import lax
from jax.experimental import pallas as pl
from jax.experimental.pallas import tpu as pltpu
from jax.experimental.pallas import tpu_sc as plsc

EMB = 128
HID = 128
H = 8
D = HID // H
N1, N2, N3 = 50000, 25000, 12500

NC, NS, LANES = 2, 16, 16
NW = NC * NS
FW = 144
NSEG = FW // LANES
BAT = 128
N1P = 50176
N2P = 25088
E1P = 200704
E2P = 102400

_SC_PARAMS = pltpu.CompilerParams(use_tc_tiling_on_sc=False)


def _emb_gather(table, idx_pad):
    per_w = N1P // NW
    sub = 224
    mesh = plsc.VectorSubcoreMesh(core_axis_name="c", subcore_axis_name="s")

    @functools.partial(
        pl.kernel, mesh=mesh,
        out_type=jax.ShapeDtypeStruct((N1P, EMB), jnp.float32),
        scratch_types=[
            pltpu.VMEM((per_w,), jnp.int32),
            pltpu.VMEM((sub, EMB), jnp.float32),
            pltpu.SemaphoreType.DMA,
        ],
        compiler_params=_SC_PARAMS,
    )
    def k(idx_hbm, table_hbm, out_hbm, idx_v, rows_v, sem):
        wid = lax.axis_index("s") * NC + lax.axis_index("c")
        base = wid * per_w
        pltpu.sync_copy(idx_hbm.at[pl.ds(base, per_w)], idx_v)
        for j in range(per_w // sub):
            pltpu.async_copy(
                table_hbm.at[idx_v.at[pl.ds(j * sub, sub)]], rows_v, sem
            ).wait()
            pltpu.sync_copy(rows_v, out_hbm.at[pl.ds(base + j * sub, sub)])

    return k(idx_pad, table)


def _prep_kernel(h_ref, w1_ref, a1_ref, wu_ref, au_ref, s_ref,
                 f1_ref, er_ref, fu_ref):
    h = h_ref[...]
    z8 = jnp.zeros((h.shape[0], 8), jnp.float32)
    f1 = jnp.dot(h, w1_ref[...], preferred_element_type=jnp.float32)
    el = jnp.dot(f1 * a1_ref[0:1, :], s_ref[...],
                 preferred_element_type=jnp.float32)
    er = jnp.dot(f1 * a1_ref[1:2, :], s_ref[...],
                 preferred_element_type=jnp.float32)
    er_ref[...] = jnp.concatenate([er, z8], axis=1)
    fu = jnp.dot(h, wu_ref[...], preferred_element_type=jnp.float32)
    elu = jnp.dot(fu * au_ref[0:1, :], s_ref[...],
                  preferred_element_type=jnp.float32)
    for seg in range(NSEG - 1):
        f1_ref[seg] = f1[:, seg * LANES:(seg + 1) * LANES]
        fu_ref[seg] = fu[:, seg * LANES:(seg + 1) * LANES]
    f1_ref[NSEG - 1] = jnp.concatenate([el, z8], axis=1)
    fu_ref[NSEG - 1] = jnp.concatenate([elu, z8], axis=1)


def _prep(h1p, w1, a1, wu, au, smat, block=1024):
    nblk = N1P // block
    return pl.pallas_call(
        _prep_kernel,
        grid=(nblk,),
        in_specs=[
            pl.BlockSpec((block, EMB), lambda i: (i, 0)),
            pl.BlockSpec((EMB, HID), lambda i: (0, 0)),
            pl.BlockSpec((2, HID), lambda i: (0, 0)),
            pl.BlockSpec((EMB, HID), lambda i: (0, 0)),
            pl.BlockSpec((2, HID), lambda i: (0, 0)),
            pl.BlockSpec((HID, 8), lambda i: (0, 0)),
        ],
        out_specs=[
            pl.BlockSpec((NSEG, block, 16), lambda i: (0, i, 0)),
            pl.BlockSpec((block, 16), lambda i: (i, 0)),
            pl.BlockSpec((NSEG, block, 16), lambda i: (0, i, 0)),
        ],
        out_shape=[
            jax.ShapeDtypeStruct((NSEG, N1P, 16), jnp.float32),
            jax.ShapeDtypeStruct((N1P, 16), jnp.float32),
            jax.ShapeDtypeStruct((NSEG, N1P, 16), jnp.float32),
        ],
    )(h1p, w1, a1, wu, au, smat)


def _sc_gat(fflat, ertab, src_pad, dst_pad, zstripe, np_dst, has_er):
    ep = src_pad.shape[0]
    epb = ep // NW
    nbat = epb // BAT
    stripe = np_dst // NS
    mesh = plsc.VectorSubcoreMesh(core_axis_name="c", subcore_axis_name="s")

    scratch = [
        pltpu.VMEM((epb,), jnp.int32),
        pltpu.VMEM((epb,), jnp.int32),
        pltpu.VMEM((BAT, LANES), jnp.float32),
        pltpu.VMEM((BAT, LANES), jnp.float32),
        pltpu.VMEM((BAT, LANES), jnp.float32),
        pltpu.VMEM((BAT,), jnp.int32),
        pltpu.VMEM((BAT,), jnp.int32),
        pltpu.VMEM((BAT,), jnp.int32),
        pltpu.VMEM_SHARED((np_dst, LANES), jnp.float32),
        pltpu.SemaphoreType.DMA,
        pltpu.SemaphoreType.DMA,
    ]

    @functools.partial(
        pl.kernel, mesh=mesh,
        out_type=[
            jax.ShapeDtypeStruct((NC, np_dst, FW), jnp.float32),
            jax.ShapeDtypeStruct((ep, LANES), jnp.float32),
        ],
        scratch_types=scratch,
        compiler_params=_SC_PARAMS,
    )
    def k(f_hbm, er_hbm, src_hbm, dst_hbm, z_hbm, out_hbm, w_hbm,
          srcv, dstv, arows, brows, wrows, gI, gI2, sI, acc, semA, semB):
        cid = lax.axis_index("c")
        sid = lax.axis_index("s")
        wid = sid * NC + cid
        ebase = wid * epb
        lane = lax.iota(jnp.int32, LANES)

        pltpu.sync_copy(src_hbm.at[pl.ds(ebase, epb)], srcv)
        pltpu.sync_copy(dst_hbm.at[pl.ds(ebase, epb)], dstv)

        def pass_body(p, _):
            pltpu.sync_copy(z_hbm, acc.at[pl.ds(sid * stripe, stripe)])
            plsc.subcore_barrier()

            def bbody(j, _):
                jb = j * BAT

                def gidx(g, _):
                    off = jb + g * LANES
                    sv = srcv[pl.ds(off, LANES)]
                    dv = dstv[pl.ds(off, LANES)]
                    gcol = jnp.where(p == 0, NSEG - 1, p - 1)
                    gI[pl.ds(g * LANES, LANES)] = gcol * N1P + sv
                    sI[pl.ds(g * LANES, LANES)] = jnp.where(
                        dv >= 0, dv, np_dst - 1)
                    return 0
                lax.fori_loop(0, BAT // LANES, gidx, 0)

                cpA = pltpu.async_copy(f_hbm.at[gI], arows, semA)
                if has_er:
                    @pl.when(p == 0)
                    def _():
                        def eidx(g, _):
                            off = jb + g * LANES
                            dv = dstv[pl.ds(off, LANES)]
                            gI2[pl.ds(g * LANES, LANES)] = jnp.where(
                                dv >= 0, dv, 0)
                            return 0
                        lax.fori_loop(0, BAT // LANES, eidx, 0)
                        pltpu.async_copy(er_hbm.at[gI2], brows, semB).wait()

                @pl.when(p > 0)
                def _():
                    pltpu.async_copy(w_hbm.at[pl.ds(ebase + jb, BAT)],
                                     wrows, semB).wait()
                cpA.wait()

                @pl.when(p == 0)
                def _():
                    def ebody(e, _):
                        elv = arows[e, pl.ds(0, LANES)]
                        sv = elv + brows[e, pl.ds(0, LANES)] if has_er else elv
                        sv = jnp.where(sv >= 0.0, sv, 0.2 * sv)
                        wrows[e, pl.ds(0, LANES)] = jnp.exp(sv)
                        return 0
                    lax.fori_loop(0, BAT, ebody, 0)
                    pltpu.sync_copy(wrows, acc.at[sI], add=True)
                    pltpu.sync_copy(wrows,
                                    w_hbm.at[pl.ds(ebase + jb, BAT)])

                for hh in range(1, NSEG):
                    @pl.when(p == hh)
                    def _(hh=hh):
                        def ebody(e, _):
                            wrow = wrows[e, pl.ds(0, LANES)]
                            frow = arows[e, pl.ds(0, LANES)]
                            arows[e, pl.ds(0, LANES)] = frow * wrow[hh - 1]
                            return 0
                        lax.fori_loop(0, BAT, ebody, 0)
                        pltpu.sync_copy(arows, acc.at[sI], add=True)
                return 0
            lax.fori_loop(0, nbat, bbody, 0)
            plsc.subcore_barrier()

            seg = jnp.where(p == 0, NSEG - 1, p - 1)
            pltpu.sync_copy(
                acc.at[pl.ds(sid * stripe, stripe)],
                out_hbm.at[cid, pl.ds(sid * stripe, stripe),
                           pl.ds(seg * LANES, LANES)])
            plsc.subcore_barrier()
            return 0
        lax.fori_loop(0, NSEG, pass_body, 0)

    return k(fflat, ertab, src_pad, dst_pad, zstripe)[0]


def _readout_kernel(*refs, nvalid, nblk, blk, with_res):
    if with_res:
        (o0_ref, o1_ref, hres_ref, rep_ref, w1_ref, b1_ref, w2_ref, b2_ref,
         bias_ref, out_ref) = refs
    else:
        (o0_ref, o1_ref, rep_ref, w1_ref, b1_ref, w2_ref, b2_ref,
         bias_ref, out_ref) = refs
        hres_ref = None
    b = pl.program_id(0)

    @pl.when(b == 0)
    def _():
        out_ref[...] = jnp.zeros_like(out_ref)

    num = o0_ref[:, :HID] + o1_ref[:, :HID]
    den8 = o0_ref[:, HID:HID + 8] + o1_ref[:, HID:HID + 8]
    den = jnp.dot(den8, rep_ref[...], preferred_element_type=jnp.float32)
    o = jnp.where(den > 0, num / jnp.where(den > 0, den, 1.0), 0.0)
    if with_res:
        o = o + hres_ref[...]
    o = o + bias_ref[...]
    t = jnp.maximum(
        jnp.dot(o, w1_ref[...], preferred_element_type=jnp.float32)
        + b1_ref[...], 0.0)
    rid = b * blk + lax.broadcasted_iota(jnp.int32, t.shape, 0)
    t = jnp.where(rid < nvalid, t, 0.0)
    out_ref[...] += jnp.sum(t, axis=0, keepdims=True)

    @pl.when(b == nblk - 1)
    def _():
        out_ref[...] = (
            jnp.dot(out_ref[...] * (1.0 / nvalid), w2_ref[...],
                    preferred_element_type=jnp.float32) + b2_ref[...])


def _readout(parts, hres, rep, p_ro, bias_row, np_dst, nvalid, blk):
    o2 = parts
    nblk = np_dst // blk
    in_specs = [
        pl.BlockSpec((blk, FW), lambda b: (b, 0)),
        pl.BlockSpec((blk, FW), lambda b: (b, 0)),
        pl.BlockSpec((blk, HID), lambda b: (b, 0)),
        pl.BlockSpec((8, HID), lambda b: (0, 0)),
        pl.BlockSpec((HID, HID), lambda b: (0, 0)),
        pl.BlockSpec((1, HID), lambda b: (0, 0)),
        pl.BlockSpec((HID, EMB), lambda b: (0, 0)),
        pl.BlockSpec((1, EMB), lambda b: (0, 0)),
        pl.BlockSpec((1, HID), lambda b: (0, 0)),
    ]
    args = [o2[0], o2[1], hres, rep, p_ro['W1'], p_ro['b1'].reshape(1, HID),
            p_ro['W2'], p_ro['b2'].reshape(1, EMB), bias_row]
    with_res = hres is not None
    if not with_res:
        in_specs.pop(2)
        args.pop(2)
    return pl.pallas_call(
        functools.partial(_readout_kernel, nvalid=nvalid, nblk=nblk, blk=blk,
                          with_res=with_res),
        grid=(nblk,),
        in_specs=in_specs,
        out_specs=pl.BlockSpec((1, EMB), lambda b: (0, 0)),
        out_shape=jax.ShapeDtypeStruct((1, EMB), jnp.float32),
    )(*args)


def _ro_row_kernel(x_ref, w1_ref, b1_ref, w2_ref, b2_ref, out_ref):
    t = jnp.maximum(
        jnp.dot(x_ref[...], w1_ref[...], preferred_element_type=jnp.float32)
        + b1_ref[...], 0.0)
    out_ref[...] = (jnp.dot(t, w2_ref[...], preferred_element_type=jnp.float32)
                    + b2_ref[...])


def _ro_row(x_row, p_ro):
    x8 = jnp.broadcast_to(x_row, (8, HID))
    out = pl.pallas_call(
        _ro_row_kernel,
        out_shape=jax.ShapeDtypeStruct((8, EMB), jnp.float32),
    )(x8, p_ro['W1'], p_ro['b1'].reshape(1, HID), p_ro['W2'],
      p_ro['b2'].reshape(1, EMB))
    return out[0:1]


def kernel(table, params, item_ids, seq1_src, seq1_dst, seq2_src, seq2_dst,
           seq3_src, seq3_dst, up1_src, up1_dst, down1_src, down1_dst,
           up2_src, up2_dst, down2_src, down2_dst):
    p1, pu = params['seq1'], params['up1']

    idx_pad = jnp.pad(item_ids.astype(jnp.int32), (0, N1P - N1))
    h1p = _emb_gather(table, idx_pad)

    a1 = jnp.stack([p1['al'].reshape(HID), p1['ar'].reshape(HID)])
    au = jnp.stack([pu['al'].reshape(HID), pu['ar'].reshape(HID)])
    smat = (jnp.arange(HID)[:, None] // D ==
            jnp.arange(8)[None, :]).astype(jnp.float32)
    f1seg, ermat, fuseg = _prep(h1p, p1['W'], a1, pu['W'], au, smat)
    f1flat = f1seg.reshape(NSEG * N1P, LANES)
    fuflat = fuseg.reshape(NSEG * N1P, LANES)

    def pad_edges(s, d, ep):
        e = s.shape[0]
        s = jnp.pad(s.astype(jnp.int32), (0, ep - e))
        d = jnp.pad(d.astype(jnp.int32), (0, ep - e), constant_values=-1)
        return s, d

    s1, d1 = pad_edges(seq1_src, seq1_dst, E1P)
    su, du = pad_edges(up1_src, up1_dst, E2P)

    z1 = jnp.zeros((N1P // NS, LANES), jnp.float32)
    z2 = jnp.zeros((N2P // NS, LANES), jnp.float32)
    parts1 = _sc_gat(f1flat, ermat, s1, d1, z1, N1P, has_er=True)
    partsu = _sc_gat(fuflat, ermat, su, du, z2, N2P, has_er=False)

    rep = (jnp.arange(8)[:, None] ==
           jnp.arange(HID)[None, :] // D).astype(jnp.float32)
    bias1 = (p1['b'] + params['down1']['b']).reshape(1, HID)
    bias2 = (params['seq2']['b'] + pu['b'] + params['down2']['b']).reshape(1, HID)

    r1 = _readout(parts1, h1p, rep, params['ro1'], bias1, N1P, N1, blk=1024)
    r2 = _readout(partsu, None, rep, params['ro2'], bias2, N2P, N2, blk=512)

    c3 = (params['seq3']['b'] + params['up2']['b']).reshape(1, HID)
    r3 = _ro_row(c3, params['ro3'])

    stack = jnp.concatenate([r1, r2, r3], axis=0)
    w = jax.nn.softmax(params['gw'])
    fused = (stack * w[:, None]).sum(axis=0)
    return fused, stack, w


# speedup vs baseline: 48.4388x; 1.0947x over previous
import functools

import jax
import jax.numpy as jnp
from jax import lax
from jax.experimental import pallas as pl
from jax.experimental.pallas import tpu as pltpu
from jax.experimental.pallas import tpu_sc as plsc

EMB = 128
HID = 128
H = 8
D = HID // H
N1, N2, N3 = 50000, 25000, 12500

NC, NS, LANES = 2, 16, 16
NW = NC * NS
FW = 144
NSEG = FW // LANES
BAT = 256
N1P = 50176
N2P = 25088
E1P = 204800
E2P = 106496

_SC_PARAMS = pltpu.CompilerParams(use_tc_tiling_on_sc=False)


def _emb_gather(table, idx_pad):
    per_w = N1P // NW
    sub = 224
    mesh = plsc.VectorSubcoreMesh(core_axis_name="c", subcore_axis_name="s")

    @functools.partial(
        pl.kernel, mesh=mesh,
        out_type=jax.ShapeDtypeStruct((N1P, EMB), jnp.float32),
        scratch_types=[
            pltpu.VMEM((per_w,), jnp.int32),
            pltpu.VMEM((sub, EMB), jnp.float32),
            pltpu.SemaphoreType.DMA,
        ],
        compiler_params=_SC_PARAMS,
    )
    def k(idx_hbm, table_hbm, out_hbm, idx_v, rows_v, sem):
        wid = lax.axis_index("s") * NC + lax.axis_index("c")
        base = wid * per_w
        pltpu.sync_copy(idx_hbm.at[pl.ds(base, per_w)], idx_v)
        for j in range(per_w // sub):
            pltpu.async_copy(
                table_hbm.at[idx_v.at[pl.ds(j * sub, sub)]], rows_v, sem
            ).wait()
            pltpu.sync_copy(rows_v, out_hbm.at[pl.ds(base + j * sub, sub)])

    return k(idx_pad, table)


def _prep_kernel(h_ref, w1_ref, a1_ref, wu_ref, au_ref, s_ref,
                 f1_ref, er_ref, fu_ref):
    h = h_ref[...]
    z8 = jnp.zeros((h.shape[0], 8), jnp.float32)
    f1 = jnp.dot(h, w1_ref[...], preferred_element_type=jnp.float32)
    el = jnp.dot(f1 * a1_ref[0:1, :], s_ref[...],
                 preferred_element_type=jnp.float32)
    er = jnp.dot(f1 * a1_ref[1:2, :], s_ref[...],
                 preferred_element_type=jnp.float32)
    er_ref[...] = jnp.concatenate([er, z8], axis=1)
    fu = jnp.dot(h, wu_ref[...], preferred_element_type=jnp.float32)
    elu = jnp.dot(fu * au_ref[0:1, :], s_ref[...],
                  preferred_element_type=jnp.float32)
    for seg in range(NSEG - 1):
        f1_ref[seg] = f1[:, seg * LANES:(seg + 1) * LANES]
        fu_ref[seg] = fu[:, seg * LANES:(seg + 1) * LANES]
    f1_ref[NSEG - 1] = jnp.concatenate([el, z8], axis=1)
    fu_ref[NSEG - 1] = jnp.concatenate([elu, z8], axis=1)


def _prep(h1p, w1, a1, wu, au, smat, block=1024):
    nblk = N1P // block
    return pl.pallas_call(
        _prep_kernel,
        grid=(nblk,),
        in_specs=[
            pl.BlockSpec((block, EMB), lambda i: (i, 0)),
            pl.BlockSpec((EMB, HID), lambda i: (0, 0)),
            pl.BlockSpec((2, HID), lambda i: (0, 0)),
            pl.BlockSpec((EMB, HID), lambda i: (0, 0)),
            pl.BlockSpec((2, HID), lambda i: (0, 0)),
            pl.BlockSpec((HID, 8), lambda i: (0, 0)),
        ],
        out_specs=[
            pl.BlockSpec((NSEG, block, 16), lambda i: (0, i, 0)),
            pl.BlockSpec((block, 16), lambda i: (i, 0)),
            pl.BlockSpec((NSEG, block, 16), lambda i: (0, i, 0)),
        ],
        out_shape=[
            jax.ShapeDtypeStruct((NSEG, N1P, 16), jnp.float32),
            jax.ShapeDtypeStruct((N1P, 16), jnp.float32),
            jax.ShapeDtypeStruct((NSEG, N1P, 16), jnp.float32),
        ],
    )(h1p, w1, a1, wu, au, smat)


def _sc_gat(fflat, ertab, src_pad, dst_pad, zstripe, np_dst, has_er):
    ep = src_pad.shape[0]
    epb = ep // NW
    nbat = epb // BAT
    stripe = np_dst // NS
    mesh = plsc.VectorSubcoreMesh(core_axis_name="c", subcore_axis_name="s")

    scratch = [
        pltpu.VMEM((epb,), jnp.int32),
        pltpu.VMEM((epb,), jnp.int32),
        pltpu.VMEM((BAT, LANES), jnp.float32),
        pltpu.VMEM((BAT, LANES), jnp.float32),
        pltpu.VMEM((BAT, LANES), jnp.float32),
        pltpu.VMEM((BAT,), jnp.int32),
        pltpu.VMEM((BAT,), jnp.int32),
        pltpu.VMEM((BAT,), jnp.int32),
        pltpu.VMEM_SHARED((np_dst, LANES), jnp.float32),
        pltpu.SemaphoreType.DMA,
        pltpu.SemaphoreType.DMA,
    ]

    @functools.partial(
        pl.kernel, mesh=mesh,
        out_type=[
            jax.ShapeDtypeStruct((NC, np_dst, FW), jnp.float32),
            jax.ShapeDtypeStruct((ep, LANES), jnp.float32),
        ],
        scratch_types=scratch,
        compiler_params=_SC_PARAMS,
    )
    def k(f_hbm, er_hbm, src_hbm, dst_hbm, z_hbm, out_hbm, w_hbm,
          srcv, dstv, arows, brows, wrows, gI, gI2, sI, acc, semA, semB):
        cid = lax.axis_index("c")
        sid = lax.axis_index("s")
        wid = sid * NC + cid
        ebase = wid * epb
        lane = lax.iota(jnp.int32, LANES)

        pltpu.sync_copy(src_hbm.at[pl.ds(ebase, epb)], srcv)
        pltpu.sync_copy(dst_hbm.at[pl.ds(ebase, epb)], dstv)

        def pass_body(p, _):
            pltpu.sync_copy(z_hbm, acc.at[pl.ds(sid * stripe, stripe)])
            plsc.subcore_barrier()

            def bbody(j, _):
                jb = j * BAT

                def gidx(g, _):
                    off = jb + g * LANES
                    sv = srcv[pl.ds(off, LANES)]
                    dv = dstv[pl.ds(off, LANES)]
                    gcol = jnp.where(p == 0, NSEG - 1, p - 1)
                    gI[pl.ds(g * LANES, LANES)] = gcol * N1P + sv
                    sI[pl.ds(g * LANES, LANES)] = jnp.where(
                        dv >= 0, dv, np_dst - 1)
                    return 0
                lax.fori_loop(0, BAT // LANES, gidx, 0)

                cpA = pltpu.async_copy(f_hbm.at[gI], arows, semA)
                if has_er:
                    @pl.when(p == 0)
                    def _():
                        def eidx(g, _):
                            off = jb + g * LANES
                            dv = dstv[pl.ds(off, LANES)]
                            gI2[pl.ds(g * LANES, LANES)] = jnp.where(
                                dv >= 0, dv, 0)
                            return 0
                        lax.fori_loop(0, BAT // LANES, eidx, 0)
                        pltpu.async_copy(er_hbm.at[gI2], brows, semB).wait()

                @pl.when(p > 0)
                def _():
                    pltpu.async_copy(w_hbm.at[pl.ds(ebase + jb, BAT)],
                                     wrows, semB).wait()
                cpA.wait()

                @pl.when(p == 0)
                def _():
                    def ebody(e4, _):
                        for kk in range(4):
                            e = e4 * 4 + kk
                            elv = arows[e, pl.ds(0, LANES)]
                            sv = (elv + brows[e, pl.ds(0, LANES)]
                                  if has_er else elv)
                            sv = jnp.where(sv >= 0.0, sv, 0.2 * sv)
                            wrows[e, pl.ds(0, LANES)] = jnp.exp(sv)
                        return 0
                    lax.fori_loop(0, BAT // 4, ebody, 0)
                    pltpu.sync_copy(wrows, acc.at[sI], add=True)
                    pltpu.sync_copy(wrows,
                                    w_hbm.at[pl.ds(ebase + jb, BAT)])

                for hh in range(1, NSEG):
                    @pl.when(p == hh)
                    def _(hh=hh):
                        def ebody(e4, _):
                            for kk in range(4):
                                e = e4 * 4 + kk
                                wrow = wrows[e, pl.ds(0, LANES)]
                                frow = arows[e, pl.ds(0, LANES)]
                                arows[e, pl.ds(0, LANES)] = frow * wrow[hh - 1]
                            return 0
                        lax.fori_loop(0, BAT // 4, ebody, 0)
                        pltpu.sync_copy(arows, acc.at[sI], add=True)
                return 0
            lax.fori_loop(0, nbat, bbody, 0)
            plsc.subcore_barrier()

            seg = jnp.where(p == 0, NSEG - 1, p - 1)
            pltpu.sync_copy(
                acc.at[pl.ds(sid * stripe, stripe)],
                out_hbm.at[cid, pl.ds(sid * stripe, stripe),
                           pl.ds(seg * LANES, LANES)])
            plsc.subcore_barrier()
            return 0
        lax.fori_loop(0, NSEG, pass_body, 0)

    return k(fflat, ertab, src_pad, dst_pad, zstripe)[0]


def _readout_kernel(*refs, nvalid, nblk, blk, with_res):
    if with_res:
        (o0_ref, o1_ref, hres_ref, rep_ref, w1_ref, b1_ref, w2_ref, b2_ref,
         bias_ref, out_ref) = refs
    else:
        (o0_ref, o1_ref, rep_ref, w1_ref, b1_ref, w2_ref, b2_ref,
         bias_ref, out_ref) = refs
        hres_ref = None
    b = pl.program_id(0)

    @pl.when(b == 0)
    def _():
        out_ref[...] = jnp.zeros_like(out_ref)

    num = o0_ref[:, :HID] + o1_ref[:, :HID]
    den8 = o0_ref[:, HID:HID + 8] + o1_ref[:, HID:HID + 8]
    den = jnp.dot(den8, rep_ref[...], preferred_element_type=jnp.float32)
    o = jnp.where(den > 0, num / jnp.where(den > 0, den, 1.0), 0.0)
    if with_res:
        o = o + hres_ref[...]
    o = o + bias_ref[...]
    t = jnp.maximum(
        jnp.dot(o, w1_ref[...], preferred_element_type=jnp.float32)
        + b1_ref[...], 0.0)
    rid = b * blk + lax.broadcasted_iota(jnp.int32, t.shape, 0)
    t = jnp.where(rid < nvalid, t, 0.0)
    out_ref[...] += jnp.sum(t, axis=0, keepdims=True)

    @pl.when(b == nblk - 1)
    def _():
        out_ref[...] = (
            jnp.dot(out_ref[...] * (1.0 / nvalid), w2_ref[...],
                    preferred_element_type=jnp.float32) + b2_ref[...])


def _readout(parts, hres, rep, p_ro, bias_row, np_dst, nvalid, blk):
    o2 = parts
    nblk = np_dst // blk
    in_specs = [
        pl.BlockSpec((blk, FW), lambda b: (b, 0)),
        pl.BlockSpec((blk, FW), lambda b: (b, 0)),
        pl.BlockSpec((blk, HID), lambda b: (b, 0)),
        pl.BlockSpec((8, HID), lambda b: (0, 0)),
        pl.BlockSpec((HID, HID), lambda b: (0, 0)),
        pl.BlockSpec((1, HID), lambda b: (0, 0)),
        pl.BlockSpec((HID, EMB), lambda b: (0, 0)),
        pl.BlockSpec((1, EMB), lambda b: (0, 0)),
        pl.BlockSpec((1, HID), lambda b: (0, 0)),
    ]
    args = [o2[0], o2[1], hres, rep, p_ro['W1'], p_ro['b1'].reshape(1, HID),
            p_ro['W2'], p_ro['b2'].reshape(1, EMB), bias_row]
    with_res = hres is not None
    if not with_res:
        in_specs.pop(2)
        args.pop(2)
    return pl.pallas_call(
        functools.partial(_readout_kernel, nvalid=nvalid, nblk=nblk, blk=blk,
                          with_res=with_res),
        grid=(nblk,),
        in_specs=in_specs,
        out_specs=pl.BlockSpec((1, EMB), lambda b: (0, 0)),
        out_shape=jax.ShapeDtypeStruct((1, EMB), jnp.float32),
    )(*args)


def _ro_row_kernel(x_ref, w1_ref, b1_ref, w2_ref, b2_ref, out_ref):
    t = jnp.maximum(
        jnp.dot(x_ref[...], w1_ref[...], preferred_element_type=jnp.float32)
        + b1_ref[...], 0.0)
    out_ref[...] = (jnp.dot(t, w2_ref[...], preferred_element_type=jnp.float32)
                    + b2_ref[...])


def _ro_row(x_row, p_ro):
    x8 = jnp.broadcast_to(x_row, (8, HID))
    out = pl.pallas_call(
        _ro_row_kernel,
        out_shape=jax.ShapeDtypeStruct((8, EMB), jnp.float32),
    )(x8, p_ro['W1'], p_ro['b1'].reshape(1, HID), p_ro['W2'],
      p_ro['b2'].reshape(1, EMB))
    return out[0:1]


def kernel(table, params, item_ids, seq1_src, seq1_dst, seq2_src, seq2_dst,
           seq3_src, seq3_dst, up1_src, up1_dst, down1_src, down1_dst,
           up2_src, up2_dst, down2_src, down2_dst):
    p1, pu = params['seq1'], params['up1']

    idx_pad = jnp.pad(item_ids.astype(jnp.int32), (0, N1P - N1))
    h1p = _emb_gather(table, idx_pad)

    a1 = jnp.stack([p1['al'].reshape(HID), p1['ar'].reshape(HID)])
    au = jnp.stack([pu['al'].reshape(HID), pu['ar'].reshape(HID)])
    smat = (jnp.arange(HID)[:, None] // D ==
            jnp.arange(8)[None, :]).astype(jnp.float32)
    f1seg, ermat, fuseg = _prep(h1p, p1['W'], a1, pu['W'], au, smat)
    f1flat = f1seg.reshape(NSEG * N1P, LANES)
    fuflat = fuseg.reshape(NSEG * N1P, LANES)

    def pad_edges(s, d, ep):
        e = s.shape[0]
        s = jnp.pad(s.astype(jnp.int32), (0, ep - e))
        d = jnp.pad(d.astype(jnp.int32), (0, ep - e), constant_values=-1)
        return s, d

    s1, d1 = pad_edges(seq1_src, seq1_dst, E1P)
    su, du = pad_edges(up1_src, up1_dst, E2P)

    z1 = jnp.zeros((N1P // NS, LANES), jnp.float32)
    z2 = jnp.zeros((N2P // NS, LANES), jnp.float32)
    parts1 = _sc_gat(f1flat, ermat, s1, d1, z1, N1P, has_er=True)
    partsu = _sc_gat(fuflat, ermat, su, du, z2, N2P, has_er=False)

    rep = (jnp.arange(8)[:, None] ==
           jnp.arange(HID)[None, :] // D).astype(jnp.float32)
    bias1 = (p1['b'] + params['down1']['b']).reshape(1, HID)
    bias2 = (params['seq2']['b'] + pu['b'] + params['down2']['b']).reshape(1, HID)

    r1 = _readout(parts1, h1p, rep, params['ro1'], bias1, N1P, N1, blk=1024)
    r2 = _readout(partsu, None, rep, params['ro2'], bias2, N2P, N2, blk=512)

    c3 = (params['seq3']['b'] + params['up2']['b']).reshape(1, HID)
    r3 = _ro_row(c3, params['ro3'])

    stack = jnp.concatenate([r1, r2, r3], axis=0)
    w = jax.nn.softmax(params['gw'])
    fused = (stack * w[:, None]).sum(axis=0)
    return fused, stack, w


# speedup vs baseline: 54.7343x; 1.1300x over previous
import functools

import jax
import jax.numpy as jnp
from jax import lax
from jax.experimental import pallas as pl
from jax.experimental.pallas import tpu as pltpu
from jax.experimental.pallas import tpu_sc as plsc

EMB = 128
HID = 128
H = 8
D = HID // H
N1, N2, N3 = 50000, 25000, 12500

NC, NS, LANES = 2, 16, 16
NW = NC * NS
FW = 144
NSEG = FW // LANES
BAT = 256
N1P = 50176
N2P = 25088
E1P = 204800
E2P = 106496

_SC_PARAMS = pltpu.CompilerParams(use_tc_tiling_on_sc=False)


def _emb_gather(table, idx_pad):
    per_w = N1P // NW
    sub = 224
    mesh = plsc.VectorSubcoreMesh(core_axis_name="c", subcore_axis_name="s")

    @functools.partial(
        pl.kernel, mesh=mesh,
        out_type=jax.ShapeDtypeStruct((N1P, EMB), jnp.float32),
        scratch_types=[
            pltpu.VMEM((per_w,), jnp.int32),
            pltpu.VMEM((sub, EMB), jnp.float32),
            pltpu.SemaphoreType.DMA,
        ],
        compiler_params=_SC_PARAMS,
    )
    def k(idx_hbm, table_hbm, out_hbm, idx_v, rows_v, sem):
        wid = lax.axis_index("s") * NC + lax.axis_index("c")
        base = wid * per_w
        pltpu.sync_copy(idx_hbm.at[pl.ds(base, per_w)], idx_v)
        for j in range(per_w // sub):
            pltpu.async_copy(
                table_hbm.at[idx_v.at[pl.ds(j * sub, sub)]], rows_v, sem
            ).wait()
            pltpu.sync_copy(rows_v, out_hbm.at[pl.ds(base + j * sub, sub)])

    return k(idx_pad, table)


def _prep_kernel(h_ref, w1_ref, a1_ref, wu_ref, au_ref, s_ref,
                 f1_ref, er_ref, fu_ref):
    h = h_ref[...]
    z8 = jnp.zeros((h.shape[0], 8), jnp.float32)
    f1 = jnp.dot(h, w1_ref[...], preferred_element_type=jnp.float32)
    el = jnp.dot(f1 * a1_ref[0:1, :], s_ref[...],
                 preferred_element_type=jnp.float32)
    er = jnp.dot(f1 * a1_ref[1:2, :], s_ref[...],
                 preferred_element_type=jnp.float32)
    er_ref[...] = jnp.concatenate([er, z8], axis=1)
    fu = jnp.dot(h, wu_ref[...], preferred_element_type=jnp.float32)
    elu = jnp.dot(fu * au_ref[0:1, :], s_ref[...],
                  preferred_element_type=jnp.float32)
    for seg in range(NSEG - 1):
        f1_ref[seg] = f1[:, seg * LANES:(seg + 1) * LANES]
        fu_ref[seg] = fu[:, seg * LANES:(seg + 1) * LANES]
    f1_ref[NSEG - 1] = jnp.concatenate([el, z8], axis=1)
    fu_ref[NSEG - 1] = jnp.concatenate([elu, z8], axis=1)


def _prep(h1p, w1, a1, wu, au, smat, block=1024):
    nblk = N1P // block
    return pl.pallas_call(
        _prep_kernel,
        grid=(nblk,),
        in_specs=[
            pl.BlockSpec((block, EMB), lambda i: (i, 0)),
            pl.BlockSpec((EMB, HID), lambda i: (0, 0)),
            pl.BlockSpec((2, HID), lambda i: (0, 0)),
            pl.BlockSpec((EMB, HID), lambda i: (0, 0)),
            pl.BlockSpec((2, HID), lambda i: (0, 0)),
            pl.BlockSpec((HID, 8), lambda i: (0, 0)),
        ],
        out_specs=[
            pl.BlockSpec((NSEG, block, 16), lambda i: (0, i, 0)),
            pl.BlockSpec((block, 16), lambda i: (i, 0)),
            pl.BlockSpec((NSEG, block, 16), lambda i: (0, i, 0)),
        ],
        out_shape=[
            jax.ShapeDtypeStruct((NSEG, N1P, 16), jnp.float32),
            jax.ShapeDtypeStruct((N1P, 16), jnp.float32),
            jax.ShapeDtypeStruct((NSEG, N1P, 16), jnp.float32),
        ],
    )(h1p, w1, a1, wu, au, smat)


def _sc_gat(fflat, ertab, src_pad, dst_pad, zstripe, np_dst, has_er):
    ep = src_pad.shape[0]
    epb = ep // NW
    nbat = epb // BAT
    stripe = np_dst // NS
    mesh = plsc.VectorSubcoreMesh(core_axis_name="c", subcore_axis_name="s")

    scratch = [
        pltpu.VMEM((epb,), jnp.int32),
        pltpu.VMEM((epb,), jnp.int32),
        pltpu.VMEM((2, BAT, LANES), jnp.float32),
        pltpu.VMEM((2, BAT, LANES), jnp.float32),
        pltpu.VMEM((2, BAT, LANES), jnp.float32),
        pltpu.VMEM((2, BAT), jnp.int32),
        pltpu.VMEM((2, BAT), jnp.int32),
        pltpu.VMEM((2, BAT), jnp.int32),
        pltpu.VMEM_SHARED((np_dst, LANES), jnp.float32),
        pltpu.SemaphoreType.DMA,
        pltpu.SemaphoreType.DMA,
        pltpu.SemaphoreType.DMA,
        pltpu.SemaphoreType.DMA,
    ]

    @functools.partial(
        pl.kernel, mesh=mesh,
        out_type=[
            jax.ShapeDtypeStruct((NC, np_dst, FW), jnp.float32),
            jax.ShapeDtypeStruct((ep, LANES), jnp.float32),
        ],
        scratch_types=scratch,
        compiler_params=_SC_PARAMS,
    )
    def k(f_hbm, er_hbm, src_hbm, dst_hbm, z_hbm, out_hbm, w_hbm,
          srcv, dstv, arows, brows, wrows, gI, gI2, sI, acc,
          semA0, semA1, semB0, semB1):
        cid = lax.axis_index("c")
        sid = lax.axis_index("s")
        wid = sid * NC + cid
        ebase = wid * epb
        lane = lax.iota(jnp.int32, LANES)
        semA = (semA0, semA1)
        semB = (semB0, semB1)
        nb2 = (nbat + 1) // 2

        pltpu.sync_copy(src_hbm.at[pl.ds(ebase, epb)], srcv)
        pltpu.sync_copy(dst_hbm.at[pl.ds(ebase, epb)], dstv)

        def pass_body(p, _):
            pltpu.sync_copy(z_hbm, acc.at[pl.ds(sid * stripe, stripe)])
            plsc.subcore_barrier()

            def fire(j, b):
                jb = j * BAT

                def gidx(g, _):
                    off = jb + g * LANES
                    sv = srcv[pl.ds(off, LANES)]
                    dv = dstv[pl.ds(off, LANES)]
                    gcol = jnp.where(p == 0, NSEG - 1, p - 1)
                    gI[b, pl.ds(g * LANES, LANES)] = gcol * N1P + sv
                    sI[b, pl.ds(g * LANES, LANES)] = jnp.where(
                        dv >= 0, dv, np_dst - 1)
                    return 0
                lax.fori_loop(0, BAT // LANES, gidx, 0)
                pltpu.async_copy(f_hbm.at[gI.at[b]], arows.at[b], semA[b])
                if has_er:
                    @pl.when(p == 0)
                    def _():
                        def eidx(g, _):
                            off = jb + g * LANES
                            dv = dstv[pl.ds(off, LANES)]
                            gI2[b, pl.ds(g * LANES, LANES)] = jnp.where(
                                dv >= 0, dv, 0)
                            return 0
                        lax.fori_loop(0, BAT // LANES, eidx, 0)
                        pltpu.async_copy(er_hbm.at[gI2.at[b]], brows.at[b],
                                         semB[b])

                @pl.when(p > 0)
                def _():
                    pltpu.async_copy(w_hbm.at[pl.ds(ebase + jb, BAT)],
                                     wrows.at[b], semB[b])

            def drain(j, b):
                jb = j * BAT
                pltpu.make_async_copy(f_hbm.at[gI.at[b]], arows.at[b],
                                      semA[b]).wait()
                if has_er:
                    @pl.when(p == 0)
                    def _():
                        pltpu.make_async_copy(er_hbm.at[gI2.at[b]],
                                              brows.at[b], semB[b]).wait()

                @pl.when(p > 0)
                def _():
                    pltpu.make_async_copy(w_hbm.at[pl.ds(ebase + jb, BAT)],
                                          wrows.at[b], semB[b]).wait()

                @pl.when(p == 0)
                def _():
                    def ebody(e4, _):
                        for kk in range(4):
                            e = e4 * 4 + kk
                            elv = arows[b, e, pl.ds(0, LANES)]
                            sv = (elv + brows[b, e, pl.ds(0, LANES)]
                                  if has_er else elv)
                            sv = jnp.where(sv >= 0.0, sv, 0.2 * sv)
                            wrows[b, e, pl.ds(0, LANES)] = jnp.exp(sv)
                        return 0
                    lax.fori_loop(0, BAT // 4, ebody, 0)
                    pltpu.sync_copy(wrows.at[b], acc.at[sI.at[b]], add=True)
                    pltpu.sync_copy(wrows.at[b],
                                    w_hbm.at[pl.ds(ebase + jb, BAT)])

                for hh in range(1, NSEG):
                    @pl.when(p == hh)
                    def _(hh=hh):
                        def ebody(e4, _):
                            for kk in range(4):
                                e = e4 * 4 + kk
                                wrow = wrows[b, e, pl.ds(0, LANES)]
                                frow = arows[b, e, pl.ds(0, LANES)]
                                arows[b, e, pl.ds(0, LANES)] = (
                                    frow * wrow[hh - 1])
                            return 0
                        lax.fori_loop(0, BAT // 4, ebody, 0)
                        pltpu.sync_copy(arows.at[b], acc.at[sI.at[b]],
                                        add=True)

            fire(0, 0)

            def bbody(jj, _):
                j0 = 2 * jj
                j1 = j0 + 1

                @pl.when(j1 < nbat)
                def _():
                    fire(j1, 1)
                drain(j0, 0)

                @pl.when(j1 < nbat)
                def _():
                    @pl.when(j0 + 2 < nbat)
                    def _():
                        fire(j0 + 2, 0)
                    drain(j1, 1)
                return 0
            lax.fori_loop(0, nb2, bbody, 0)
            plsc.subcore_barrier()

            seg = jnp.where(p == 0, NSEG - 1, p - 1)
            pltpu.sync_copy(
                acc.at[pl.ds(sid * stripe, stripe)],
                out_hbm.at[cid, pl.ds(sid * stripe, stripe),
                           pl.ds(seg * LANES, LANES)])
            plsc.subcore_barrier()
            return 0
        lax.fori_loop(0, NSEG, pass_body, 0)

    return k(fflat, ertab, src_pad, dst_pad, zstripe)[0]


def _readout_kernel(*refs, nvalid, nblk, blk, with_res):
    if with_res:
        (o0_ref, o1_ref, hres_ref, rep_ref, w1_ref, b1_ref, w2_ref, b2_ref,
         bias_ref, out_ref) = refs
    else:
        (o0_ref, o1_ref, rep_ref, w1_ref, b1_ref, w2_ref, b2_ref,
         bias_ref, out_ref) = refs
        hres_ref = None
    b = pl.program_id(0)

    @pl.when(b == 0)
    def _():
        out_ref[...] = jnp.zeros_like(out_ref)

    num = o0_ref[:, :HID] + o1_ref[:, :HID]
    den8 = o0_ref[:, HID:HID + 8] + o1_ref[:, HID:HID + 8]
    den = jnp.dot(den8, rep_ref[...], preferred_element_type=jnp.float32)
    o = jnp.where(den > 0, num / jnp.where(den > 0, den, 1.0), 0.0)
    if with_res:
        o = o + hres_ref[...]
    o = o + bias_ref[...]
    t = jnp.maximum(
        jnp.dot(o, w1_ref[...], preferred_element_type=jnp.float32)
        + b1_ref[...], 0.0)
    rid = b * blk + lax.broadcasted_iota(jnp.int32, t.shape, 0)
    t = jnp.where(rid < nvalid, t, 0.0)
    out_ref[...] += jnp.sum(t, axis=0, keepdims=True)

    @pl.when(b == nblk - 1)
    def _():
        out_ref[...] = (
            jnp.dot(out_ref[...] * (1.0 / nvalid), w2_ref[...],
                    preferred_element_type=jnp.float32) + b2_ref[...])


def _readout(parts, hres, rep, p_ro, bias_row, np_dst, nvalid, blk):
    o2 = parts
    nblk = np_dst // blk
    in_specs = [
        pl.BlockSpec((blk, FW), lambda b: (b, 0)),
        pl.BlockSpec((blk, FW), lambda b: (b, 0)),
        pl.BlockSpec((blk, HID), lambda b: (b, 0)),
        pl.BlockSpec((8, HID), lambda b: (0, 0)),
        pl.BlockSpec((HID, HID), lambda b: (0, 0)),
        pl.BlockSpec((1, HID), lambda b: (0, 0)),
        pl.BlockSpec((HID, EMB), lambda b: (0, 0)),
        pl.BlockSpec((1, EMB), lambda b: (0, 0)),
        pl.BlockSpec((1, HID), lambda b: (0, 0)),
    ]
    args = [o2[0], o2[1], hres, rep, p_ro['W1'], p_ro['b1'].reshape(1, HID),
            p_ro['W2'], p_ro['b2'].reshape(1, EMB), bias_row]
    with_res = hres is not None
    if not with_res:
        in_specs.pop(2)
        args.pop(2)
    return pl.pallas_call(
        functools.partial(_readout_kernel, nvalid=nvalid, nblk=nblk, blk=blk,
                          with_res=with_res),
        grid=(nblk,),
        in_specs=in_specs,
        out_specs=pl.BlockSpec((1, EMB), lambda b: (0, 0)),
        out_shape=jax.ShapeDtypeStruct((1, EMB), jnp.float32),
    )(*args)


def _ro_row_kernel(x_ref, w1_ref, b1_ref, w2_ref, b2_ref, out_ref):
    t = jnp.maximum(
        jnp.dot(x_ref[...], w1_ref[...], preferred_element_type=jnp.float32)
        + b1_ref[...], 0.0)
    out_ref[...] = (jnp.dot(t, w2_ref[...], preferred_element_type=jnp.float32)
                    + b2_ref[...])


def _ro_row(x_row, p_ro):
    x8 = jnp.broadcast_to(x_row, (8, HID))
    out = pl.pallas_call(
        _ro_row_kernel,
        out_shape=jax.ShapeDtypeStruct((8, EMB), jnp.float32),
    )(x8, p_ro['W1'], p_ro['b1'].reshape(1, HID), p_ro['W2'],
      p_ro['b2'].reshape(1, EMB))
    return out[0:1]


def kernel(table, params, item_ids, seq1_src, seq1_dst, seq2_src, seq2_dst,
           seq3_src, seq3_dst, up1_src, up1_dst, down1_src, down1_dst,
           up2_src, up2_dst, down2_src, down2_dst):
    p1, pu = params['seq1'], params['up1']

    idx_pad = jnp.pad(item_ids.astype(jnp.int32), (0, N1P - N1))
    h1p = _emb_gather(table, idx_pad)

    a1 = jnp.stack([p1['al'].reshape(HID), p1['ar'].reshape(HID)])
    au = jnp.stack([pu['al'].reshape(HID), pu['ar'].reshape(HID)])
    smat = (jnp.arange(HID)[:, None] // D ==
            jnp.arange(8)[None, :]).astype(jnp.float32)
    f1seg, ermat, fuseg = _prep(h1p, p1['W'], a1, pu['W'], au, smat)
    f1flat = f1seg.reshape(NSEG * N1P, LANES)
    fuflat = fuseg.reshape(NSEG * N1P, LANES)

    def pad_edges(s, d, ep):
        e = s.shape[0]
        s = jnp.pad(s.astype(jnp.int32), (0, ep - e))
        d = jnp.pad(d.astype(jnp.int32), (0, ep - e), constant_values=-1)
        return s, d

    s1, d1 = pad_edges(seq1_src, seq1_dst, E1P)
    su, du = pad_edges(up1_src, up1_dst, E2P)

    z1 = jnp.zeros((N1P // NS, LANES), jnp.float32)
    z2 = jnp.zeros((N2P // NS, LANES), jnp.float32)
    parts1 = _sc_gat(f1flat, ermat, s1, d1, z1, N1P, has_er=True)
    partsu = _sc_gat(fuflat, ermat, su, du, z2, N2P, has_er=False)

    rep = (jnp.arange(8)[:, None] ==
           jnp.arange(HID)[None, :] // D).astype(jnp.float32)
    bias1 = (p1['b'] + params['down1']['b']).reshape(1, HID)
    bias2 = (params['seq2']['b'] + pu['b'] + params['down2']['b']).reshape(1, HID)

    r1 = _readout(parts1, h1p, rep, params['ro1'], bias1, N1P, N1, blk=1024)
    r2 = _readout(partsu, None, rep, params['ro2'], bias2, N2P, N2, blk=512)

    c3 = (params['seq3']['b'] + params['up2']['b']).reshape(1, HID)
    r3 = _ro_row(c3, params['ro3'])

    stack = jnp.concatenate([r1, r2, r3], axis=0)
    w = jax.nn.softmax(params['gw'])
    fused = (stack * w[:, None]).sum(axis=0)
    return fused, stack, w


# speedup vs baseline: 56.1776x; 1.0264x over previous
import functools

import jax
import jax.numpy as jnp
from jax import lax
from jax.experimental import pallas as pl
from jax.experimental.pallas import tpu as pltpu
from jax.experimental.pallas import tpu_sc as plsc

EMB = 128
HID = 128
H = 8
D = HID // H
N1, N2, N3 = 50000, 25000, 12500

NC, NS, LANES = 2, 16, 16
NW = NC * NS
FW = 144
NSEG = FW // LANES
BAT = 256
N1P = 50176
N2P = 25088
E1P = 204800
E2P = 106496

_SC_PARAMS = pltpu.CompilerParams(use_tc_tiling_on_sc=False)


def _emb_gather(table, idx_pad):
    per_w = N1P // NW
    sub = 224
    mesh = plsc.VectorSubcoreMesh(core_axis_name="c", subcore_axis_name="s")

    @functools.partial(
        pl.kernel, mesh=mesh,
        out_type=jax.ShapeDtypeStruct((N1P, EMB), jnp.float32),
        scratch_types=[
            pltpu.VMEM((per_w,), jnp.int32),
            pltpu.VMEM((sub, EMB), jnp.float32),
            pltpu.SemaphoreType.DMA,
        ],
        compiler_params=_SC_PARAMS,
    )
    def k(idx_hbm, table_hbm, out_hbm, idx_v, rows_v, sem):
        wid = lax.axis_index("s") * NC + lax.axis_index("c")
        base = wid * per_w
        pltpu.sync_copy(idx_hbm.at[pl.ds(base, per_w)], idx_v)
        for j in range(per_w // sub):
            pltpu.async_copy(
                table_hbm.at[idx_v.at[pl.ds(j * sub, sub)]], rows_v, sem
            ).wait()
            pltpu.sync_copy(rows_v, out_hbm.at[pl.ds(base + j * sub, sub)])

    return k(idx_pad, table)


def _prep_kernel(h_ref, w1_ref, a1_ref, wu_ref, au_ref, s_ref,
                 f1_ref, er_ref, fu_ref):
    h = h_ref[...]
    z8 = jnp.zeros((h.shape[0], 8), jnp.float32)
    f1 = jnp.dot(h, w1_ref[...], preferred_element_type=jnp.float32)
    el = jnp.dot(f1 * a1_ref[0:1, :], s_ref[...],
                 preferred_element_type=jnp.float32)
    er = jnp.dot(f1 * a1_ref[1:2, :], s_ref[...],
                 preferred_element_type=jnp.float32)
    er_ref[...] = jnp.concatenate([er, z8], axis=1)
    fu = jnp.dot(h, wu_ref[...], preferred_element_type=jnp.float32)
    elu = jnp.dot(fu * au_ref[0:1, :], s_ref[...],
                  preferred_element_type=jnp.float32)
    for seg in range(NSEG - 1):
        f1_ref[seg] = f1[:, seg * LANES:(seg + 1) * LANES]
        fu_ref[seg] = fu[:, seg * LANES:(seg + 1) * LANES]
    f1_ref[NSEG - 1] = jnp.concatenate([el, z8], axis=1)
    fu_ref[NSEG - 1] = jnp.concatenate([elu, z8], axis=1)


def _prep(h1p, w1, a1, wu, au, smat, block=1024):
    nblk = N1P // block
    return pl.pallas_call(
        _prep_kernel,
        grid=(nblk,),
        in_specs=[
            pl.BlockSpec((block, EMB), lambda i: (i, 0)),
            pl.BlockSpec((EMB, HID), lambda i: (0, 0)),
            pl.BlockSpec((2, HID), lambda i: (0, 0)),
            pl.BlockSpec((EMB, HID), lambda i: (0, 0)),
            pl.BlockSpec((2, HID), lambda i: (0, 0)),
            pl.BlockSpec((HID, 8), lambda i: (0, 0)),
        ],
        out_specs=[
            pl.BlockSpec((NSEG, block, 16), lambda i: (0, i, 0)),
            pl.BlockSpec((block, 16), lambda i: (i, 0)),
            pl.BlockSpec((NSEG, block, 16), lambda i: (0, i, 0)),
        ],
        out_shape=[
            jax.ShapeDtypeStruct((NSEG, N1P, 16), jnp.float32),
            jax.ShapeDtypeStruct((N1P, 16), jnp.float32),
            jax.ShapeDtypeStruct((NSEG, N1P, 16), jnp.float32),
        ],
    )(h1p, w1, a1, wu, au, smat)


def _sc_gat(fflat, ertab, src_pad, dst_pad, zstripe, np_dst, has_er):
    ep = src_pad.shape[0]
    epb = ep // NW
    nbat = epb // BAT
    stripe = np_dst // NS
    mesh = plsc.VectorSubcoreMesh(core_axis_name="c", subcore_axis_name="s")

    scratch = [
        pltpu.VMEM((epb,), jnp.int32),
        pltpu.VMEM((epb,), jnp.int32),
        pltpu.VMEM((2, BAT, LANES), jnp.float32),
        pltpu.VMEM((2, BAT, LANES), jnp.float32),
        pltpu.VMEM((2, BAT, LANES), jnp.float32),
        pltpu.VMEM((2, BAT), jnp.int32),
        pltpu.VMEM((2, BAT), jnp.int32),
        pltpu.VMEM((2, BAT), jnp.int32),
        pltpu.VMEM_SHARED((np_dst, LANES), jnp.float32),
        pltpu.SemaphoreType.DMA,
        pltpu.SemaphoreType.DMA,
        pltpu.SemaphoreType.DMA,
        pltpu.SemaphoreType.DMA,
        pltpu.SemaphoreType.DMA,
        pltpu.SemaphoreType.DMA,
    ]

    @functools.partial(
        pl.kernel, mesh=mesh,
        out_type=[
            jax.ShapeDtypeStruct((NC, np_dst, FW), jnp.float32),
            jax.ShapeDtypeStruct((ep, LANES), jnp.float32),
        ],
        scratch_types=scratch,
        compiler_params=_SC_PARAMS,
    )
    def k(f_hbm, er_hbm, src_hbm, dst_hbm, z_hbm, out_hbm, w_hbm,
          srcv, dstv, arows, brows, wrows, gI, gI2, sI, acc,
          semA0, semA1, semB0, semB1, semS0, semS1):
        cid = lax.axis_index("c")
        sid = lax.axis_index("s")
        wid = sid * NC + cid
        ebase = wid * epb
        lane = lax.iota(jnp.int32, LANES)
        semA = (semA0, semA1)
        semB = (semB0, semB1)
        semS = (semS0, semS1)
        nb2 = (nbat + 1) // 2

        pltpu.sync_copy(src_hbm.at[pl.ds(ebase, epb)], srcv)
        pltpu.sync_copy(dst_hbm.at[pl.ds(ebase, epb)], dstv)

        def pass_body(p, _):
            pltpu.sync_copy(z_hbm, acc.at[pl.ds(sid * stripe, stripe)])
            plsc.subcore_barrier()

            def fire(j, b):
                jb = j * BAT

                @pl.when(j >= 2)
                def _():
                    @pl.when(p == 0)
                    def _():
                        pltpu.make_async_copy(
                            wrows.at[b], acc.at[sI.at[b]], semS[b]).wait()

                    @pl.when(p > 0)
                    def _():
                        pltpu.make_async_copy(
                            arows.at[b], acc.at[sI.at[b]], semS[b]).wait()

                def gidx(g, _):
                    off = jb + g * LANES
                    sv = srcv[pl.ds(off, LANES)]
                    dv = dstv[pl.ds(off, LANES)]
                    gcol = jnp.where(p == 0, NSEG - 1, p - 1)
                    gI[b, pl.ds(g * LANES, LANES)] = gcol * N1P + sv
                    sI[b, pl.ds(g * LANES, LANES)] = jnp.where(
                        dv >= 0, dv, np_dst - 1)
                    return 0
                lax.fori_loop(0, BAT // LANES, gidx, 0)
                pltpu.async_copy(f_hbm.at[gI.at[b]], arows.at[b], semA[b])
                if has_er:
                    @pl.when(p == 0)
                    def _():
                        def eidx(g, _):
                            off = jb + g * LANES
                            dv = dstv[pl.ds(off, LANES)]
                            gI2[b, pl.ds(g * LANES, LANES)] = jnp.where(
                                dv >= 0, dv, 0)
                            return 0
                        lax.fori_loop(0, BAT // LANES, eidx, 0)
                        pltpu.async_copy(er_hbm.at[gI2.at[b]], brows.at[b],
                                         semB[b])

                @pl.when(p > 0)
                def _():
                    pltpu.async_copy(w_hbm.at[pl.ds(ebase + jb, BAT)],
                                     wrows.at[b], semB[b])

            def drain(j, b):
                jb = j * BAT
                pltpu.make_async_copy(f_hbm.at[gI.at[b]], arows.at[b],
                                      semA[b]).wait()
                if has_er:
                    @pl.when(p == 0)
                    def _():
                        pltpu.make_async_copy(er_hbm.at[gI2.at[b]],
                                              brows.at[b], semB[b]).wait()

                @pl.when(p > 0)
                def _():
                    pltpu.make_async_copy(w_hbm.at[pl.ds(ebase + jb, BAT)],
                                          wrows.at[b], semB[b]).wait()

                @pl.when(p == 0)
                def _():
                    def ebody(e4, _):
                        for kk in range(4):
                            e = e4 * 4 + kk
                            elv = arows[b, e, pl.ds(0, LANES)]
                            sv = (elv + brows[b, e, pl.ds(0, LANES)]
                                  if has_er else elv)
                            sv = jnp.where(sv >= 0.0, sv, 0.2 * sv)
                            wrows[b, e, pl.ds(0, LANES)] = jnp.exp(sv)
                        return 0
                    lax.fori_loop(0, BAT // 4, ebody, 0)
                    pltpu.async_copy(wrows.at[b], acc.at[sI.at[b]], semS[b],
                                     add=True)
                    pltpu.sync_copy(wrows.at[b],
                                    w_hbm.at[pl.ds(ebase + jb, BAT)])

                for hh in range(1, NSEG):
                    @pl.when(p == hh)
                    def _(hh=hh):
                        def ebody(e4, _):
                            for kk in range(4):
                                e = e4 * 4 + kk
                                wrow = wrows[b, e, pl.ds(0, LANES)]
                                frow = arows[b, e, pl.ds(0, LANES)]
                                arows[b, e, pl.ds(0, LANES)] = (
                                    frow * wrow[hh - 1])
                            return 0
                        lax.fori_loop(0, BAT // 4, ebody, 0)
                        pltpu.async_copy(arows.at[b], acc.at[sI.at[b]],
                                         semS[b], add=True)

            fire(0, 0)

            def bbody(jj, _):
                j0 = 2 * jj
                j1 = j0 + 1

                @pl.when(j1 < nbat)
                def _():
                    fire(j1, 1)
                drain(j0, 0)

                @pl.when(j1 < nbat)
                def _():
                    @pl.when(j0 + 2 < nbat)
                    def _():
                        fire(j0 + 2, 0)
                    drain(j1, 1)
                return 0
            lax.fori_loop(0, nb2, bbody, 0)
            for b in range(2):
                @pl.when(p == 0)
                def _(b=b):
                    pltpu.make_async_copy(
                        wrows.at[b], acc.at[sI.at[b]], semS[b]).wait()

                @pl.when(p > 0)
                def _(b=b):
                    pltpu.make_async_copy(
                        arows.at[b], acc.at[sI.at[b]], semS[b]).wait()
            plsc.subcore_barrier()

            seg = jnp.where(p == 0, NSEG - 1, p - 1)
            pltpu.sync_copy(
                acc.at[pl.ds(sid * stripe, stripe)],
                out_hbm.at[cid, pl.ds(sid * stripe, stripe),
                           pl.ds(seg * LANES, LANES)])
            return 0
        lax.fori_loop(0, NSEG, pass_body, 0)

    return k(fflat, ertab, src_pad, dst_pad, zstripe)[0]


def _readout_kernel(*refs, nvalid, nblk, blk, with_res):
    if with_res:
        (o0_ref, o1_ref, hres_ref, rep_ref, w1_ref, b1_ref, w2_ref, b2_ref,
         bias_ref, out_ref) = refs
    else:
        (o0_ref, o1_ref, rep_ref, w1_ref, b1_ref, w2_ref, b2_ref,
         bias_ref, out_ref) = refs
        hres_ref = None
    b = pl.program_id(0)

    @pl.when(b == 0)
    def _():
        out_ref[...] = jnp.zeros_like(out_ref)

    num = o0_ref[:, :HID] + o1_ref[:, :HID]
    den8 = o0_ref[:, HID:HID + 8] + o1_ref[:, HID:HID + 8]
    den = jnp.dot(den8, rep_ref[...], preferred_element_type=jnp.float32)
    o = jnp.where(den > 0, num / jnp.where(den > 0, den, 1.0), 0.0)
    if with_res:
        o = o + hres_ref[...]
    o = o + bias_ref[...]
    t = jnp.maximum(
        jnp.dot(o, w1_ref[...], preferred_element_type=jnp.float32)
        + b1_ref[...], 0.0)
    rid = b * blk + lax.broadcasted_iota(jnp.int32, t.shape, 0)
    t = jnp.where(rid < nvalid, t, 0.0)
    out_ref[...] += jnp.sum(t, axis=0, keepdims=True)

    @pl.when(b == nblk - 1)
    def _():
        out_ref[...] = (
            jnp.dot(out_ref[...] * (1.0 / nvalid), w2_ref[...],
                    preferred_element_type=jnp.float32) + b2_ref[...])


def _readout(parts, hres, rep, p_ro, bias_row, np_dst, nvalid, blk):
    o2 = parts
    nblk = np_dst // blk
    in_specs = [
        pl.BlockSpec((blk, FW), lambda b: (b, 0)),
        pl.BlockSpec((blk, FW), lambda b: (b, 0)),
        pl.BlockSpec((blk, HID), lambda b: (b, 0)),
        pl.BlockSpec((8, HID), lambda b: (0, 0)),
        pl.BlockSpec((HID, HID), lambda b: (0, 0)),
        pl.BlockSpec((1, HID), lambda b: (0, 0)),
        pl.BlockSpec((HID, EMB), lambda b: (0, 0)),
        pl.BlockSpec((1, EMB), lambda b: (0, 0)),
        pl.BlockSpec((1, HID), lambda b: (0, 0)),
    ]
    args = [o2[0], o2[1], hres, rep, p_ro['W1'], p_ro['b1'].reshape(1, HID),
            p_ro['W2'], p_ro['b2'].reshape(1, EMB), bias_row]
    with_res = hres is not None
    if not with_res:
        in_specs.pop(2)
        args.pop(2)
    return pl.pallas_call(
        functools.partial(_readout_kernel, nvalid=nvalid, nblk=nblk, blk=blk,
                          with_res=with_res),
        grid=(nblk,),
        in_specs=in_specs,
        out_specs=pl.BlockSpec((1, EMB), lambda b: (0, 0)),
        out_shape=jax.ShapeDtypeStruct((1, EMB), jnp.float32),
    )(*args)


def _ro_row_kernel(x_ref, w1_ref, b1_ref, w2_ref, b2_ref, out_ref):
    t = jnp.maximum(
        jnp.dot(x_ref[...], w1_ref[...], preferred_element_type=jnp.float32)
        + b1_ref[...], 0.0)
    out_ref[...] = (jnp.dot(t, w2_ref[...], preferred_element_type=jnp.float32)
                    + b2_ref[...])


def _ro_row(x_row, p_ro):
    x8 = jnp.broadcast_to(x_row, (8, HID))
    out = pl.pallas_call(
        _ro_row_kernel,
        out_shape=jax.ShapeDtypeStruct((8, EMB), jnp.float32),
    )(x8, p_ro['W1'], p_ro['b1'].reshape(1, HID), p_ro['W2'],
      p_ro['b2'].reshape(1, EMB))
    return out[0:1]


def kernel(table, params, item_ids, seq1_src, seq1_dst, seq2_src, seq2_dst,
           seq3_src, seq3_dst, up1_src, up1_dst, down1_src, down1_dst,
           up2_src, up2_dst, down2_src, down2_dst):
    p1, pu = params['seq1'], params['up1']

    idx_pad = jnp.pad(item_ids.astype(jnp.int32), (0, N1P - N1))
    h1p = _emb_gather(table, idx_pad)

    a1 = jnp.stack([p1['al'].reshape(HID), p1['ar'].reshape(HID)])
    au = jnp.stack([pu['al'].reshape(HID), pu['ar'].reshape(HID)])
    smat = (jnp.arange(HID)[:, None] // D ==
            jnp.arange(8)[None, :]).astype(jnp.float32)
    f1seg, ermat, fuseg = _prep(h1p, p1['W'], a1, pu['W'], au, smat)
    f1flat = f1seg.reshape(NSEG * N1P, LANES)
    fuflat = fuseg.reshape(NSEG * N1P, LANES)

    def pad_edges(s, d, ep):
        e = s.shape[0]
        s = jnp.pad(s.astype(jnp.int32), (0, ep - e))
        d = jnp.pad(d.astype(jnp.int32), (0, ep - e), constant_values=-1)
        return s, d

    s1, d1 = pad_edges(seq1_src, seq1_dst, E1P)
    su, du = pad_edges(up1_src, up1_dst, E2P)

    z1 = jnp.zeros((N1P // NS, LANES), jnp.float32)
    z2 = jnp.zeros((N2P // NS, LANES), jnp.float32)
    parts1 = _sc_gat(f1flat, ermat, s1, d1, z1, N1P, has_er=True)
    partsu = _sc_gat(fuflat, ermat, su, du, z2, N2P, has_er=False)

    rep = (jnp.arange(8)[:, None] ==
           jnp.arange(HID)[None, :] // D).astype(jnp.float32)
    bias1 = (p1['b'] + params['down1']['b']).reshape(1, HID)
    bias2 = (params['seq2']['b'] + pu['b'] + params['down2']['b']).reshape(1, HID)

    r1 = _readout(parts1, h1p, rep, params['ro1'], bias1, N1P, N1, blk=1024)
    r2 = _readout(partsu, None, rep, params['ro2'], bias2, N2P, N2, blk=512)

    c3 = (params['seq3']['b'] + params['up2']['b']).reshape(1, HID)
    r3 = _ro_row(c3, params['ro3'])

    stack = jnp.concatenate([r1, r2, r3], axis=0)
    w = jax.nn.softmax(params['gw'])
    fused = (stack * w[:, None]).sum(axis=0)
    return fused, stack, w
